# baseline (speedup 1.0000x reference)
import jax
import jax.numpy as jnp
import numpy as np
from jax import lax
from jax.experimental import pallas as pl
from jax.experimental.pallas import tpu as pltpu
from jax.experimental.pallas import tpu_sc as plsc

H, W = 512, 512
N_JUNC = 300
NUM_SAMPLES = 64
N_PAIRS = N_JUNC * (N_JUNC - 1) // 2
N_TILES = 32
PAIRS_PER_TILE = 1408
N_GROUPS = PAIRS_PER_TILE // 16
CHUNK_LO = (0, 171, 342)
CHUNK_HI = (171, 342, 512)
CHUNK_ROWS = 176
CHUNK_COLS = W + 4
TABLE_WORDS = CHUNK_ROWS * CHUNK_COLS
OUT_FLAT = 90304
DUMP_CELL = 90000

_PATCH_OFFS = tuple(
    (oh, ow)
    for oh in (-2, -1, 0, 1, 2)
    for ow in (-2, -1, 0, 1, 2)
    if oh * oh + ow * ow <= 4
)


_I0, _I1 = np.triu_indices(N_JUNC, k=1)
_PAD_CELLS = (np.arange(N_TILES * PAIRS_PER_TILE - N_PAIRS, dtype=np.int32)
              % 304 + DUMP_CELL)


def _sc_body(chunks_hbm, fields_hbm, tu_hbm, lohi_hbm, oidx_hbm, out_hbm,
             table_v, fields_v, acc_v, tu_v, lohi_v, oidx_v):
    cid = lax.axis_index("c")
    sid = lax.axis_index("s")
    wid = sid * 2 + cid

    pltpu.sync_copy(fields_hbm.at[wid], fields_v)
    pltpu.sync_copy(tu_hbm, tu_v)
    pltpu.sync_copy(lohi_hbm, lohi_v)
    pltpu.sync_copy(oidx_hbm.at[wid], oidx_v)

    zeros16 = jnp.zeros((16,), jnp.float32)

    def zero_body(g, carry):
        acc_v[g, :] = zeros16
        return carry

    lax.fori_loop(0, N_GROUPS, zero_body, 0)

    def pass_body(c, carry):
        pltpu.sync_copy(chunks_hbm.at[c], table_v)
        lovec = lohi_v[c, :]
        hivec = lohi_v[c + 3, :]

        lof = tu_v[2 * NUM_SAMPLES + c, :]
        hif = tu_v[2 * NUM_SAMPLES + 3 + c, :]

        @plsc.parallel_loop(0, N_GROUPS)
        def group_body(g):
            sh = fields_v[g, 0, :]
            sw = fields_v[g, 1, :]
            eh = fields_v[g, 2, :]
            ew = fields_v[g, 3, :]
            th2 = fields_v[g, 4, :]
            acc0 = acc_v[g, :]

            denom = sh - eh
            inv = 1.0 / denom
            ta = (lof - eh) * inv
            tb = (hif - eh) * inv
            degen = denom == 0.0
            tlo = jnp.where(degen, 0.0, jnp.minimum(ta, tb))
            thi = jnp.where(degen, 1.0, jnp.maximum(ta, tb))
            empty = (thi < 0.0) | (tlo > 1.0)
            tlo_c = jnp.clip(tlo, 0.0, 1.0)
            thi_c = jnp.clip(thi, 0.0, 1.0)
            slo_l = jnp.clip((tlo_c * 63.0).astype(jnp.int32) - 1, 0, NUM_SAMPLES)
            shi_l = jnp.clip((thi_c * 63.0).astype(jnp.int32) + 2, 0, NUM_SAMPLES)
            slo_l = jnp.where(empty, NUM_SAMPLES, slo_l)
            shi_l = jnp.where(empty, 0, shi_l)
            slo_g = jnp.min(slo_l, axis=0)
            shi_g = jnp.max(shi_l, axis=0)

            def sbody(s, acc):
                if True:
                    t = tu_v[s, :]
                    u = tu_v[s + NUM_SAMPLES, :]
                    h = jnp.clip(sh * t + eh * u, 0.0, float(H - 1))
                    w = jnp.clip(sw * t + ew * u, 0.0, float(W - 1))
                    rh = (h + 0.5).astype(jnp.int32)
                    rhf = rh.astype(jnp.float32)
                    fix_h = ((rhf - h) == 0.5) & ((rh & 1) == 1)
                    rh = rh - jnp.where(fix_h, 1, 0)
                    rhf = rh.astype(jnp.float32)
                    rw = (w + 0.5).astype(jnp.int32)
                    rwf = rw.astype(jnp.float32)
                    fix_w = ((rwf - w) == 0.5) & ((rw & 1) == 1)
                    rw = rw - jnp.where(fix_w, 1, 0)
                    rwf = rw.astype(jnp.float32)
                    fh = h - rhf
                    fw = w - rwf
                    owner = (rh >= lovec) & (rh < hivec)
                    th2e = jnp.where(owner, th2, -1.0)
                    rbase = rh + 2 - lovec
                    rterm = {}
                    cloc = {}
                    dh2 = {}
                    dw2 = {}
                    for k in (-2, -1, 0, 1, 2):
                        rterm[k] = jnp.clip(rbase + k, 0, CHUNK_ROWS - 1) * CHUNK_COLS
                        cloc[k] = rw + (k + 2)
                        dh = fh - float(k)
                        dw = fw - float(k)
                        dh2[k] = dh * dh
                        dw2[k] = dw * dw
                    vs = []
                    for (oh, ow) in _PATCH_OFFS:
                        v = plsc.load_gather(table_v, [rterm[oh] + cloc[ow]])
                        vf = plsc.bitcast(v, jnp.float32)
                        d2 = dh2[oh] + dw2[ow]
                        vs.append(jnp.where(d2 < th2e, vf, 0.0))
                    while len(vs) > 1:
                        vs = [jnp.maximum(a, b) for a, b in zip(vs[::2], vs[1::2])] + (
                            [vs[-1]] if len(vs) % 2 else [])
                    acc = acc + vs[0]
                return acc

            acc = lax.fori_loop(slo_g, shi_g, sbody, acc0)
            acc_v[g, :] = acc

        return carry

    lax.fori_loop(0, 3, pass_body, 0)

    zi16 = jnp.zeros((16,), jnp.int32)

    def zmap_body(g, carry):
        table_v[pl.ds(g * 16, 16)] = zi16
        return carry

    lax.fori_loop(0, OUT_FLAT // 16, zmap_body, 0)

    for g in range(N_GROUPS):
        det = jnp.where(acc_v[g, :] > 32.0, 1, 0).astype(jnp.int32)
        idx_a = oidx_v[pl.ds(g * 16, 16)]
        idx_b = oidx_v[pl.ds(PAIRS_PER_TILE + g * 16, 16)]
        plsc.store_scatter(table_v, [idx_a], det)
        plsc.store_scatter(table_v, [idx_b], det)

    pltpu.sync_copy(table_v.at[pl.ds(0, OUT_FLAT)], out_hbm.at[wid])


@jax.jit
def kernel(junctions, heatmap):
    junctions = junctions.astype(jnp.float32)
    heatmap = heatmap.astype(jnp.float32)

    cand_start = junctions[_I0]
    cand_end = junctions[_I1]
    seg_len = jnp.sqrt(jnp.sum((cand_start - cand_end) ** 2, axis=-1))
    norm_len = seg_len / (H ** 2 + W ** 2) ** 0.5
    dist_thresh = 0.5 * (2.0 ** 0.5) + 2.0 * norm_len
    th2 = dist_thresh * dist_thresh

    key = ((cand_end[:, 0].astype(jnp.int32) >> 4) * 32
           + (cand_start[:, 0].astype(jnp.int32) >> 4))
    perm = jnp.argsort(key)
    cand_start = cand_start[perm]
    cand_end = cand_end[perm]
    th2 = th2[perm]
    i0_p = jnp.asarray(_I0.astype(np.int32))[perm]
    i1_p = jnp.asarray(_I1.astype(np.int32))[perm]
    cells_a = jnp.concatenate([i0_p * N_JUNC + i1_p, jnp.asarray(_PAD_CELLS)])
    cells_b = jnp.concatenate([i1_p * N_JUNC + i0_p, jnp.asarray(_PAD_CELLS)])
    oidx = jnp.concatenate(
        [cells_a.reshape(N_TILES, PAIRS_PER_TILE),
         cells_b.reshape(N_TILES, PAIRS_PER_TILE)], axis=1)

    fields = jnp.stack(
        [cand_start[:, 0], cand_start[:, 1], cand_end[:, 0], cand_end[:, 1],
         th2, jnp.zeros_like(th2)], axis=1)
    fields = jnp.pad(fields, ((0, N_TILES * PAIRS_PER_TILE - N_PAIRS), (0, 0)))
    fields_hbm = fields.reshape(N_TILES, N_GROUPS, 16, 6).transpose(0, 1, 3, 2)

    t = jnp.linspace(0.0, 1.0, NUM_SAMPLES).astype(jnp.float32)
    u = (1.0 - t).astype(jnp.float32)
    lohif = np.zeros((6,), np.float32)
    for c in range(3):
        lohif[c] = -1e9 if CHUNK_LO[c] == 0 else CHUNK_LO[c] - 0.6
        lohif[c + 3] = 1e9 if CHUNK_HI[c] == H else CHUNK_HI[c] - 0.4
    tu = (jnp.concatenate([t, u, jnp.asarray(lohif)])[:, None]
          * jnp.ones((1, 16), jnp.float32))

    lohi = np.zeros((6, 16), np.int32)
    for c in range(3):
        lohi[c, :] = CHUNK_LO[c]
        lohi[c + 3, :] = CHUNK_HI[c]
    lohi = jnp.asarray(lohi)

    ridx = np.clip(np.asarray(CHUNK_LO)[:, None] - 2 + np.arange(CHUNK_ROWS)[None, :],
                   0, H - 1)
    rows = heatmap[ridx]
    chunks = jnp.concatenate(
        [rows[:, :, :1], rows[:, :, :1], rows, rows[:, :, -1:], rows[:, :, -1:]],
        axis=2).reshape(3, TABLE_WORDS)
    chunks = lax.bitcast_convert_type(chunks, jnp.int32)

    out_parts = pl.kernel(
        _sc_body,
        out_type=jax.ShapeDtypeStruct((N_TILES, OUT_FLAT), jnp.int32),
        mesh=plsc.VectorSubcoreMesh(core_axis_name="c", subcore_axis_name="s"),
        compiler_params=pltpu.CompilerParams(
            needs_layout_passes=False, use_tc_tiling_on_sc=False),
        scratch_types=[
            pltpu.VMEM((TABLE_WORDS,), jnp.int32),
            pltpu.VMEM((N_GROUPS, 6, 16), jnp.float32),
            pltpu.VMEM((N_GROUPS, 16), jnp.float32),
            pltpu.VMEM((2 * NUM_SAMPLES + 6, 16), jnp.float32),
            pltpu.VMEM((6, 16), jnp.int32),
            pltpu.VMEM((2 * PAIRS_PER_TILE,), jnp.int32),
        ],
    )(chunks, fields_hbm, tu, lohi, oidx)

    line_map = jnp.sum(out_parts, axis=0)[:N_JUNC * N_JUNC].reshape(N_JUNC, N_JUNC)
    return line_map, junctions, heatmap


# speedup vs baseline: 1.2289x; 1.2289x over previous
import jax
import jax.numpy as jnp
import numpy as np
from jax import lax
from jax.experimental import pallas as pl
from jax.experimental.pallas import tpu as pltpu
from jax.experimental.pallas import tpu_sc as plsc

H, W = 512, 512
N_JUNC = 300
NUM_SAMPLES = 64
N_PAIRS = N_JUNC * (N_JUNC - 1) // 2
N_TILES = 32
PAIRS_PER_TILE = 1408
N_GROUPS = PAIRS_PER_TILE // 16
CHUNK_LO = (0, 171, 342)
CHUNK_HI = (171, 342, 512)
CHUNK_ROWS = 176
CHUNK_COLS = W + 4
TABLE_WORDS = CHUNK_ROWS * CHUNK_COLS
OUT_FLAT = 90304
DUMP_CELL = 90000

_PATCH_OFFS = tuple(
    (oh, ow)
    for oh in (-2, -1, 0, 1, 2)
    for ow in (-2, -1, 0, 1, 2)
    if oh * oh + ow * ow <= 4
)


_I0, _I1 = np.triu_indices(N_JUNC, k=1)
_PAD_CELLS = (np.arange(N_TILES * PAIRS_PER_TILE - N_PAIRS, dtype=np.int32)
              % 304 + DUMP_CELL)


def _sc_body(chunks_hbm, fields_hbm, tu_hbm, lohi_hbm, oidx_hbm, out_hbm,
             table_v, fields_v, acc_v, tu_v, lohi_v, oidx_v):
    cid = lax.axis_index("c")
    sid = lax.axis_index("s")
    wid = sid * 2 + cid

    pltpu.sync_copy(fields_hbm.at[wid], fields_v)
    pltpu.sync_copy(tu_hbm, tu_v)
    pltpu.sync_copy(lohi_hbm, lohi_v)
    pltpu.sync_copy(oidx_hbm.at[wid], oidx_v)

    zeros16 = jnp.zeros((16,), jnp.float32)

    def zero_body(g, carry):
        acc_v[g, :] = zeros16
        return carry

    lax.fori_loop(0, N_GROUPS, zero_body, 0)

    def pass_body(c, carry):
        pltpu.sync_copy(chunks_hbm.at[c], table_v)
        lovec = lohi_v[c, :]
        hivec = lohi_v[c + 3, :]

        lof = tu_v[2 * NUM_SAMPLES + c, :]
        hif = tu_v[2 * NUM_SAMPLES + 3 + c, :]

        @plsc.parallel_loop(0, N_GROUPS)
        def group_body(g):
            sh = fields_v[g, 0, :]
            sw = fields_v[g, 1, :]
            eh = fields_v[g, 2, :]
            ew = fields_v[g, 3, :]
            th2 = fields_v[g, 4, :]
            acc0 = acc_v[g, :]

            denom = sh - eh
            inv = 1.0 / denom
            ta = (lof - eh) * inv
            tb = (hif - eh) * inv
            degen = denom == 0.0
            tlo = jnp.where(degen, 0.0, jnp.minimum(ta, tb))
            thi = jnp.where(degen, 1.0, jnp.maximum(ta, tb))
            empty = (thi < 0.0) | (tlo > 1.0)
            tlo_c = jnp.clip(tlo, 0.0, 1.0)
            thi_c = jnp.clip(thi, 0.0, 1.0)
            slo_l = jnp.clip((tlo_c * 63.0).astype(jnp.int32) - 1, 0, NUM_SAMPLES)
            shi_l = jnp.clip((thi_c * 63.0).astype(jnp.int32) + 2, 0, NUM_SAMPLES)
            slo_l = jnp.where(empty, NUM_SAMPLES, slo_l)
            shi_l = jnp.where(empty, 0, shi_l)
            slo_g = jnp.min(slo_l, axis=0)
            shi_g = jnp.max(shi_l, axis=0)

            def sbody(s, acc):
                if True:
                    t = tu_v[s, :]
                    u = tu_v[s + NUM_SAMPLES, :]
                    h = jnp.clip(sh * t + eh * u, 0.0, float(H - 1))
                    w = jnp.clip(sw * t + ew * u, 0.0, float(W - 1))
                    rh = (h + 0.5).astype(jnp.int32)
                    rhf = rh.astype(jnp.float32)
                    fix_h = ((rhf - h) == 0.5) & ((rh & 1) == 1)
                    rh = rh - jnp.where(fix_h, 1, 0)
                    rhf = rh.astype(jnp.float32)
                    rw = (w + 0.5).astype(jnp.int32)
                    rwf = rw.astype(jnp.float32)
                    fix_w = ((rwf - w) == 0.5) & ((rw & 1) == 1)
                    rw = rw - jnp.where(fix_w, 1, 0)
                    rwf = rw.astype(jnp.float32)
                    fh = h - rhf
                    fw = w - rwf
                    owner = (rh >= lovec) & (rh < hivec)
                    th2e = jnp.where(owner, th2, -1.0)
                    rbase = rh + 2 - lovec
                    rterm = {}
                    cloc = {}
                    dh2 = {}
                    dw2 = {}
                    for k in (-2, -1, 0, 1, 2):
                        rterm[k] = jnp.clip(rbase + k, 0, CHUNK_ROWS - 1) * CHUNK_COLS
                        cloc[k] = rw + (k + 2)
                        dh = fh - float(k)
                        dw = fw - float(k)
                        dh2[k] = dh * dh
                        dw2[k] = dw * dw
                    vs = []
                    for (oh, ow) in _PATCH_OFFS:
                        v = plsc.load_gather(table_v, [rterm[oh] + cloc[ow]])
                        vf = plsc.bitcast(v, jnp.float32)
                        d2 = dh2[oh] + dw2[ow]
                        vs.append(jnp.where(d2 < th2e, vf, 0.0))
                    while len(vs) > 1:
                        vs = [jnp.maximum(a, b) for a, b in zip(vs[::2], vs[1::2])] + (
                            [vs[-1]] if len(vs) % 2 else [])
                    acc = acc + vs[0]
                return acc

            acc = lax.fori_loop(slo_g, shi_g, sbody, acc0)
            acc_v[g, :] = acc

        return carry

    lax.fori_loop(0, 3, pass_body, 0)

    zi16 = jnp.zeros((16,), jnp.int32)

    def zmap_body(g, carry):
        table_v[pl.ds(g * 16, 16)] = zi16
        return carry

    lax.fori_loop(0, OUT_FLAT // 16, zmap_body, 0)

    for g in range(N_GROUPS):
        det = jnp.where(acc_v[g, :] > 32.0, 1, 0).astype(jnp.int32)
        idx_a = oidx_v[pl.ds(g * 16, 16)]
        idx_b = oidx_v[pl.ds(PAIRS_PER_TILE + g * 16, 16)]
        plsc.store_scatter(table_v, [idx_a], det)
        plsc.store_scatter(table_v, [idx_b], det)

    pltpu.sync_copy(table_v.at[pl.ds(0, OUT_FLAT)], out_hbm.at[wid])


@jax.jit
def kernel(junctions, heatmap):
    junctions = junctions.astype(jnp.float32)
    heatmap = heatmap.astype(jnp.float32)

    jorder = jnp.argsort(junctions[:, 0]).astype(jnp.int32)
    pa = jorder[_I0]
    pb = jorder[_I1]
    o0 = jnp.minimum(pa, pb)
    o1 = jnp.maximum(pa, pb)
    cand_start = junctions[o0]
    cand_end = junctions[o1]
    seg_len = jnp.sqrt(jnp.sum((cand_start - cand_end) ** 2, axis=-1))
    norm_len = seg_len / (H ** 2 + W ** 2) ** 0.5
    dist_thresh = 0.5 * (2.0 ** 0.5) + 2.0 * norm_len
    th2 = dist_thresh * dist_thresh

    cells_a = jnp.concatenate([o0 * N_JUNC + o1, jnp.asarray(_PAD_CELLS)])
    cells_b = jnp.concatenate([o1 * N_JUNC + o0, jnp.asarray(_PAD_CELLS)])
    oidx = jnp.concatenate(
        [cells_a.reshape(N_TILES, PAIRS_PER_TILE),
         cells_b.reshape(N_TILES, PAIRS_PER_TILE)], axis=1)

    fields = jnp.stack(
        [cand_start[:, 0], cand_start[:, 1], cand_end[:, 0], cand_end[:, 1],
         th2, jnp.zeros_like(th2)], axis=1)
    fields = jnp.pad(fields, ((0, N_TILES * PAIRS_PER_TILE - N_PAIRS), (0, 0)))
    fields_hbm = fields.reshape(N_TILES, N_GROUPS, 16, 6).transpose(0, 1, 3, 2)

    t = jnp.linspace(0.0, 1.0, NUM_SAMPLES).astype(jnp.float32)
    u = (1.0 - t).astype(jnp.float32)
    lohif = np.zeros((6,), np.float32)
    for c in range(3):
        lohif[c] = -1e9 if CHUNK_LO[c] == 0 else CHUNK_LO[c] - 0.6
        lohif[c + 3] = 1e9 if CHUNK_HI[c] == H else CHUNK_HI[c] - 0.4
    tu = (jnp.concatenate([t, u, jnp.asarray(lohif)])[:, None]
          * jnp.ones((1, 16), jnp.float32))

    lohi = np.zeros((6, 16), np.int32)
    for c in range(3):
        lohi[c, :] = CHUNK_LO[c]
        lohi[c + 3, :] = CHUNK_HI[c]
    lohi = jnp.asarray(lohi)

    ridx = np.clip(np.asarray(CHUNK_LO)[:, None] - 2 + np.arange(CHUNK_ROWS)[None, :],
                   0, H - 1)
    rows = heatmap[ridx]
    chunks = jnp.concatenate(
        [rows[:, :, :1], rows[:, :, :1], rows, rows[:, :, -1:], rows[:, :, -1:]],
        axis=2).reshape(3, TABLE_WORDS)
    chunks = lax.bitcast_convert_type(chunks, jnp.int32)

    out_parts = pl.kernel(
        _sc_body,
        out_type=jax.ShapeDtypeStruct((N_TILES, OUT_FLAT), jnp.int32),
        mesh=plsc.VectorSubcoreMesh(core_axis_name="c", subcore_axis_name="s"),
        compiler_params=pltpu.CompilerParams(
            needs_layout_passes=False, use_tc_tiling_on_sc=False),
        scratch_types=[
            pltpu.VMEM((TABLE_WORDS,), jnp.int32),
            pltpu.VMEM((N_GROUPS, 6, 16), jnp.float32),
            pltpu.VMEM((N_GROUPS, 16), jnp.float32),
            pltpu.VMEM((2 * NUM_SAMPLES + 6, 16), jnp.float32),
            pltpu.VMEM((6, 16), jnp.int32),
            pltpu.VMEM((2 * PAIRS_PER_TILE,), jnp.int32),
        ],
    )(chunks, fields_hbm, tu, lohi, oidx)

    line_map = jnp.sum(out_parts, axis=0)[:N_JUNC * N_JUNC].reshape(N_JUNC, N_JUNC)
    return line_map, junctions, heatmap


# speedup vs baseline: 1.2322x; 1.0027x over previous
import jax
import jax.numpy as jnp
import numpy as np
from jax import lax
from jax.experimental import pallas as pl
from jax.experimental.pallas import tpu as pltpu
from jax.experimental.pallas import tpu_sc as plsc

H, W = 512, 512
N_JUNC = 300
NUM_SAMPLES = 64
N_PAIRS = N_JUNC * (N_JUNC - 1) // 2
N_TILES = 32
PAIRS_PER_TILE = 1408
N_GROUPS = PAIRS_PER_TILE // 16
CHUNK_LO = (0, 171, 342)
CHUNK_HI = (171, 342, 512)
CHUNK_ROWS = 176
CHUNK_COLS = W + 4
TABLE_WORDS = CHUNK_ROWS * CHUNK_COLS
OUT_FLAT = 90304
DUMP_CELL = 90000

_PATCH_OFFS = tuple(
    (oh, ow)
    for oh in (-2, -1, 0, 1, 2)
    for ow in (-2, -1, 0, 1, 2)
    if oh * oh + ow * ow <= 4
)


_I0, _I1 = np.triu_indices(N_JUNC, k=1)
_PAD_CELLS = (np.arange(N_TILES * PAIRS_PER_TILE - N_PAIRS, dtype=np.int32)
              % 304 + DUMP_CELL)


def _sc_body(chunks_hbm, fields_hbm, tu_hbm, lohi_hbm, oidx_hbm, out_hbm,
             table_v, fields_v, acc_v, tu_v, lohi_v, oidx_v):
    cid = lax.axis_index("c")
    sid = lax.axis_index("s")
    wid = sid * 2 + cid

    pltpu.sync_copy(fields_hbm.at[wid], fields_v)
    pltpu.sync_copy(tu_hbm, tu_v)
    pltpu.sync_copy(lohi_hbm, lohi_v)
    pltpu.sync_copy(oidx_hbm.at[wid], oidx_v)

    zeros16 = jnp.zeros((16,), jnp.float32)

    def zero_body(g, carry):
        acc_v[g, :] = zeros16
        return carry

    lax.fori_loop(0, N_GROUPS, zero_body, 0)

    def pass_body(c, carry):
        pltpu.sync_copy(chunks_hbm.at[c], table_v)
        lovec = lohi_v[c, :]
        hivec = lohi_v[c + 3, :]

        lof = tu_v[2 * NUM_SAMPLES + c, :]
        hif = tu_v[2 * NUM_SAMPLES + 3 + c, :]

        @plsc.parallel_loop(0, N_GROUPS)
        def group_body(g):
            sh = fields_v[g, 0, :]
            sw = fields_v[g, 1, :]
            eh = fields_v[g, 2, :]
            ew = fields_v[g, 3, :]
            th2 = fields_v[g, 4, :]
            acc0 = acc_v[g, :]

            denom = sh - eh
            inv = 1.0 / denom
            ta = (lof - eh) * inv
            tb = (hif - eh) * inv
            degen = denom == 0.0
            tlo = jnp.where(degen, 0.0, jnp.minimum(ta, tb))
            thi = jnp.where(degen, 1.0, jnp.maximum(ta, tb))
            empty = (thi < 0.0) | (tlo > 1.0)
            tlo_c = jnp.clip(tlo, 0.0, 1.0)
            thi_c = jnp.clip(thi, 0.0, 1.0)
            slo_l = jnp.clip((tlo_c * 63.0).astype(jnp.int32) - 1, 0, NUM_SAMPLES)
            shi_l = jnp.clip((thi_c * 63.0).astype(jnp.int32) + 2, 0, NUM_SAMPLES)
            slo_l = jnp.where(empty, NUM_SAMPLES, slo_l)
            shi_l = jnp.where(empty, 0, shi_l)
            slo_g = jnp.min(slo_l, axis=0)
            shi_g = jnp.max(shi_l, axis=0)

            def sbody(s, acc):
                if True:
                    t = tu_v[s, :]
                    u = tu_v[s + NUM_SAMPLES, :]
                    h = jnp.clip(sh * t + eh * u, 0.0, float(H - 1))
                    w = jnp.clip(sw * t + ew * u, 0.0, float(W - 1))
                    rh = (h + 0.5).astype(jnp.int32)
                    rhf = rh.astype(jnp.float32)
                    fix_h = ((rhf - h) == 0.5) & ((rh & 1) == 1)
                    rh = rh - jnp.where(fix_h, 1, 0)
                    rhf = rh.astype(jnp.float32)
                    rw = (w + 0.5).astype(jnp.int32)
                    rwf = rw.astype(jnp.float32)
                    fix_w = ((rwf - w) == 0.5) & ((rw & 1) == 1)
                    rw = rw - jnp.where(fix_w, 1, 0)
                    rwf = rw.astype(jnp.float32)
                    fh = h - rhf
                    fw = w - rwf
                    owner = (rh >= lovec) & (rh < hivec)
                    th2e = jnp.where(owner, th2, -1.0)
                    rbase = rh + 2 - lovec
                    rterm = {}
                    cloc = {}
                    dh2 = {}
                    dw2 = {}
                    for k in (-2, -1, 0, 1, 2):
                        rterm[k] = jnp.clip(rbase + k, 0, CHUNK_ROWS - 1) * CHUNK_COLS
                        cloc[k] = rw + (k + 2)
                        dh = fh - float(k)
                        dw = fw - float(k)
                        dh2[k] = dh * dh
                        dw2[k] = dw * dw
                    vs = []
                    for (oh, ow) in _PATCH_OFFS:
                        v = plsc.load_gather(table_v, [rterm[oh] + cloc[ow]])
                        vf = plsc.bitcast(v, jnp.float32)
                        d2 = dh2[oh] + dw2[ow]
                        vs.append(jnp.where(d2 < th2e, vf, 0.0))
                    while len(vs) > 1:
                        vs = [jnp.maximum(a, b) for a, b in zip(vs[::2], vs[1::2])] + (
                            [vs[-1]] if len(vs) % 2 else [])
                    acc = acc + vs[0]
                return acc

            acc = lax.fori_loop(slo_g, shi_g, sbody, acc0)
            acc_v[g, :] = acc

        return carry

    lax.fori_loop(0, 3, pass_body, 0)

    zi16 = jnp.zeros((16,), jnp.int32)

    def zmap_body(g, carry):
        table_v[pl.ds(g * 16, 16)] = zi16
        return carry

    lax.fori_loop(0, OUT_FLAT // 16, zmap_body, 0)

    for g in range(N_GROUPS):
        det = jnp.where(acc_v[g, :] > 32.0, 1, 0).astype(jnp.int32)
        idx_a = oidx_v[pl.ds(g * 16, 16)]
        idx_b = oidx_v[pl.ds(PAIRS_PER_TILE + g * 16, 16)]
        plsc.store_scatter(table_v, [idx_a], det)
        plsc.store_scatter(table_v, [idx_b], det)

    pltpu.sync_copy(table_v.at[pl.ds(0, OUT_FLAT)], out_hbm.at[wid])


@jax.jit
def kernel(junctions, heatmap):
    junctions = junctions.astype(jnp.float32)
    heatmap = heatmap.astype(jnp.float32)

    jh = junctions[:, 0]
    iota = jnp.arange(N_JUNC, dtype=jnp.int32)
    less = (jh[None, :] < jh[:, None]) | (
        (jh[None, :] == jh[:, None]) & (iota[None, :] < iota[:, None]))
    rank = jnp.sum(less.astype(jnp.int32), axis=1)
    onehot = (rank[:, None] == iota[None, :]).astype(jnp.int32)
    jorder = jnp.sum(onehot * iota[:, None], axis=0).astype(jnp.int32)
    pa = jorder[_I0]
    pb = jorder[_I1]
    o0 = jnp.minimum(pa, pb)
    o1 = jnp.maximum(pa, pb)
    cand_start = junctions[o0]
    cand_end = junctions[o1]
    seg_len = jnp.sqrt(jnp.sum((cand_start - cand_end) ** 2, axis=-1))
    norm_len = seg_len / (H ** 2 + W ** 2) ** 0.5
    dist_thresh = 0.5 * (2.0 ** 0.5) + 2.0 * norm_len
    th2 = dist_thresh * dist_thresh

    cells_a = jnp.concatenate([o0 * N_JUNC + o1, jnp.asarray(_PAD_CELLS)])
    cells_b = jnp.concatenate([o1 * N_JUNC + o0, jnp.asarray(_PAD_CELLS)])
    oidx = jnp.concatenate(
        [cells_a.reshape(N_TILES, PAIRS_PER_TILE),
         cells_b.reshape(N_TILES, PAIRS_PER_TILE)], axis=1)

    fields = jnp.stack(
        [cand_start[:, 0], cand_start[:, 1], cand_end[:, 0], cand_end[:, 1],
         th2, jnp.zeros_like(th2)], axis=1)
    fields = jnp.pad(fields, ((0, N_TILES * PAIRS_PER_TILE - N_PAIRS), (0, 0)))
    fields_hbm = fields.reshape(N_TILES, N_GROUPS, 16, 6).transpose(0, 1, 3, 2)

    t = jnp.linspace(0.0, 1.0, NUM_SAMPLES).astype(jnp.float32)
    u = (1.0 - t).astype(jnp.float32)
    lohif = np.zeros((6,), np.float32)
    for c in range(3):
        lohif[c] = -1e9 if CHUNK_LO[c] == 0 else CHUNK_LO[c] - 0.6
        lohif[c + 3] = 1e9 if CHUNK_HI[c] == H else CHUNK_HI[c] - 0.4
    tu = (jnp.concatenate([t, u, jnp.asarray(lohif)])[:, None]
          * jnp.ones((1, 16), jnp.float32))

    lohi = np.zeros((6, 16), np.int32)
    for c in range(3):
        lohi[c, :] = CHUNK_LO[c]
        lohi[c + 3, :] = CHUNK_HI[c]
    lohi = jnp.asarray(lohi)

    ridx = np.clip(np.asarray(CHUNK_LO)[:, None] - 2 + np.arange(CHUNK_ROWS)[None, :],
                   0, H - 1)
    rows = heatmap[ridx]
    chunks = jnp.concatenate(
        [rows[:, :, :1], rows[:, :, :1], rows, rows[:, :, -1:], rows[:, :, -1:]],
        axis=2).reshape(3, TABLE_WORDS)
    chunks = lax.bitcast_convert_type(chunks, jnp.int32)

    out_parts = pl.kernel(
        _sc_body,
        out_type=jax.ShapeDtypeStruct((N_TILES, OUT_FLAT), jnp.int32),
        mesh=plsc.VectorSubcoreMesh(core_axis_name="c", subcore_axis_name="s"),
        compiler_params=pltpu.CompilerParams(
            needs_layout_passes=False, use_tc_tiling_on_sc=False),
        scratch_types=[
            pltpu.VMEM((TABLE_WORDS,), jnp.int32),
            pltpu.VMEM((N_GROUPS, 6, 16), jnp.float32),
            pltpu.VMEM((N_GROUPS, 16), jnp.float32),
            pltpu.VMEM((2 * NUM_SAMPLES + 6, 16), jnp.float32),
            pltpu.VMEM((6, 16), jnp.int32),
            pltpu.VMEM((2 * PAIRS_PER_TILE,), jnp.int32),
        ],
    )(chunks, fields_hbm, tu, lohi, oidx)

    line_map = jnp.sum(out_parts, axis=0)[:N_JUNC * N_JUNC].reshape(N_JUNC, N_JUNC)
    return line_map, junctions, heatmap


# speedup vs baseline: 2.0299x; 1.6474x over previous
import jax
import jax.numpy as jnp
import numpy as np
from jax import lax
from jax.experimental import pallas as pl
from jax.experimental.pallas import tpu as pltpu
from jax.experimental.pallas import tpu_sc as plsc

H, W = 512, 512
N_JUNC = 300
NUM_SAMPLES = 64
N_PAIRS = N_JUNC * (N_JUNC - 1) // 2
N_TILES = 32
PAIRS_PER_TILE = 1408
N_GROUPS = PAIRS_PER_TILE // 16
CHUNK_LO = (0, 171, 342)
CHUNK_HI = (171, 342, 512)
CHUNK_ROWS = 176
CHUNK_COLS = W + 4
TABLE_WORDS = CHUNK_ROWS * CHUNK_COLS
OUT_FLAT = 90304
DUMP_CELL = 90000

_PATCH_OFFS = tuple(
    (oh, ow)
    for oh in (-2, -1, 0, 1, 2)
    for ow in (-2, -1, 0, 1, 2)
    if oh * oh + ow * ow <= 4
)


_I0, _I1 = np.triu_indices(N_JUNC, k=1)
_PAD_CELLS = (np.arange(N_TILES * PAIRS_PER_TILE - N_PAIRS, dtype=np.int32)
              % 304 + DUMP_CELL)


def _sc_body(chunks_hbm, fields_hbm, tu_hbm, lohi_hbm, oidx_hbm, out_hbm,
             table_v, fields_v, acc_v, tu_v, lohi_v, oidx_v):
    cid = lax.axis_index("c")
    sid = lax.axis_index("s")
    wid = sid * 2 + cid

    pltpu.sync_copy(fields_hbm.at[wid], fields_v)
    pltpu.sync_copy(tu_hbm, tu_v)
    pltpu.sync_copy(lohi_hbm, lohi_v)
    pltpu.sync_copy(oidx_hbm.at[wid], oidx_v)

    zeros16 = jnp.zeros((16,), jnp.float32)

    def zero_body(g, carry):
        acc_v[g, :] = zeros16
        return carry

    lax.fori_loop(0, N_GROUPS, zero_body, 0)

    def pass_body(c, carry):
        pltpu.sync_copy(chunks_hbm.at[c], table_v)
        lovec = lohi_v[c, :]
        hivec = lohi_v[c + 3, :]

        lof = tu_v[2 * NUM_SAMPLES + c, :]
        hif = tu_v[2 * NUM_SAMPLES + 3 + c, :]

        @plsc.parallel_loop(0, N_GROUPS)
        def group_body(g):
            sh = fields_v[g, 0, :]
            sw = fields_v[g, 1, :]
            eh = fields_v[g, 2, :]
            ew = fields_v[g, 3, :]
            th2 = fields_v[g, 4, :]
            acc0 = acc_v[g, :]

            denom = sh - eh
            inv = 1.0 / denom
            ta = (lof - eh) * inv
            tb = (hif - eh) * inv
            degen = denom == 0.0
            tlo = jnp.where(degen, 0.0, jnp.minimum(ta, tb))
            thi = jnp.where(degen, 1.0, jnp.maximum(ta, tb))
            empty = (thi < 0.0) | (tlo > 1.0)
            tlo_c = jnp.clip(tlo, 0.0, 1.0)
            thi_c = jnp.clip(thi, 0.0, 1.0)
            slo_l = jnp.clip((tlo_c * 63.0).astype(jnp.int32) - 1, 0, NUM_SAMPLES)
            shi_l = jnp.clip((thi_c * 63.0).astype(jnp.int32) + 2, 0, NUM_SAMPLES)
            slo_l = jnp.where(empty, NUM_SAMPLES, slo_l)
            shi_l = jnp.where(empty, 0, shi_l)
            slo_g = jnp.min(slo_l, axis=0)
            shi_g = jnp.max(shi_l, axis=0)

            def sbody(s, acc):
                if True:
                    t = tu_v[s, :]
                    u = tu_v[s + NUM_SAMPLES, :]
                    h = jnp.clip(sh * t + eh * u, 0.0, float(H - 1))
                    w = jnp.clip(sw * t + ew * u, 0.0, float(W - 1))
                    rh = (h + 0.5).astype(jnp.int32)
                    rhf = rh.astype(jnp.float32)
                    fix_h = ((rhf - h) == 0.5) & ((rh & 1) == 1)
                    rh = rh - jnp.where(fix_h, 1, 0)
                    rhf = rh.astype(jnp.float32)
                    rw = (w + 0.5).astype(jnp.int32)
                    rwf = rw.astype(jnp.float32)
                    fix_w = ((rwf - w) == 0.5) & ((rw & 1) == 1)
                    rw = rw - jnp.where(fix_w, 1, 0)
                    rwf = rw.astype(jnp.float32)
                    fh = h - rhf
                    fw = w - rwf
                    owner = (rh >= lovec) & (rh < hivec)
                    th2e = jnp.where(owner, th2, -1.0)
                    rbase = rh + 2 - lovec
                    rterm = {}
                    cloc = {}
                    dh2 = {}
                    dw2 = {}
                    for k in (-2, -1, 0, 1, 2):
                        rterm[k] = jnp.clip(rbase + k, 0, CHUNK_ROWS - 1) * CHUNK_COLS
                        cloc[k] = rw + (k + 2)
                        dh = fh - float(k)
                        dw = fw - float(k)
                        dh2[k] = dh * dh
                        dw2[k] = dw * dw
                    vs = []
                    for (oh, ow) in _PATCH_OFFS:
                        v = plsc.load_gather(table_v, [rterm[oh] + cloc[ow]])
                        vf = plsc.bitcast(v, jnp.float32)
                        d2 = dh2[oh] + dw2[ow]
                        vs.append(jnp.where(d2 < th2e, vf, 0.0))
                    while len(vs) > 1:
                        vs = [jnp.maximum(a, b) for a, b in zip(vs[::2], vs[1::2])] + (
                            [vs[-1]] if len(vs) % 2 else [])
                    acc = acc + vs[0]
                return acc

            acc = lax.fori_loop(slo_g, shi_g, sbody, acc0)
            acc_v[g, :] = acc

        return carry

    lax.fori_loop(0, 3, pass_body, 0)

    zi16 = jnp.zeros((16,), jnp.int32)

    def zmap_body(g, carry):
        table_v[pl.ds(g * 16, 16)] = zi16
        return carry

    lax.fori_loop(0, OUT_FLAT // 16, zmap_body, 0)

    for g in range(N_GROUPS):
        det = jnp.where(acc_v[g, :] > 32.0, 1, 0).astype(jnp.int32)
        idx_a = oidx_v[pl.ds(g * 16, 16)]
        idx_b = oidx_v[pl.ds(PAIRS_PER_TILE + g * 16, 16)]
        plsc.store_scatter(table_v, [idx_a], det)
        plsc.store_scatter(table_v, [idx_b], det)

    pltpu.sync_copy(table_v.at[pl.ds(0, OUT_FLAT)], out_hbm.at[wid])


@jax.jit
def kernel(junctions, heatmap):
    junctions = junctions.astype(jnp.float32)
    heatmap = heatmap.astype(jnp.float32)

    o0 = jnp.asarray(_I0.astype(np.int32))
    o1 = jnp.asarray(_I1.astype(np.int32))
    cand_start = junctions[o0]
    cand_end = junctions[o1]
    seg_len = jnp.sqrt(jnp.sum((cand_start - cand_end) ** 2, axis=-1))
    norm_len = seg_len / (H ** 2 + W ** 2) ** 0.5
    dist_thresh = 0.5 * (2.0 ** 0.5) + 2.0 * norm_len
    th2 = dist_thresh * dist_thresh

    cells_a = jnp.concatenate([o0 * N_JUNC + o1, jnp.asarray(_PAD_CELLS)])
    cells_b = jnp.concatenate([o1 * N_JUNC + o0, jnp.asarray(_PAD_CELLS)])
    oidx = jnp.concatenate(
        [cells_a.reshape(N_TILES, PAIRS_PER_TILE),
         cells_b.reshape(N_TILES, PAIRS_PER_TILE)], axis=1)

    fields = jnp.stack(
        [cand_start[:, 0], cand_start[:, 1], cand_end[:, 0], cand_end[:, 1],
         th2, jnp.zeros_like(th2)], axis=1)
    fields = jnp.pad(fields, ((0, N_TILES * PAIRS_PER_TILE - N_PAIRS), (0, 0)))
    fields_hbm = fields.reshape(N_TILES, N_GROUPS, 16, 6).transpose(0, 1, 3, 2)

    t = jnp.linspace(0.0, 1.0, NUM_SAMPLES).astype(jnp.float32)
    u = (1.0 - t).astype(jnp.float32)
    lohif = np.zeros((6,), np.float32)
    for c in range(3):
        lohif[c] = -1e9 if CHUNK_LO[c] == 0 else CHUNK_LO[c] - 0.6
        lohif[c + 3] = 1e9 if CHUNK_HI[c] == H else CHUNK_HI[c] - 0.4
    tu = (jnp.concatenate([t, u, jnp.asarray(lohif)])[:, None]
          * jnp.ones((1, 16), jnp.float32))

    lohi = np.zeros((6, 16), np.int32)
    for c in range(3):
        lohi[c, :] = CHUNK_LO[c]
        lohi[c + 3, :] = CHUNK_HI[c]
    lohi = jnp.asarray(lohi)

    ridx = np.clip(np.asarray(CHUNK_LO)[:, None] - 2 + np.arange(CHUNK_ROWS)[None, :],
                   0, H - 1)
    rows = heatmap[ridx]
    chunks = jnp.concatenate(
        [rows[:, :, :1], rows[:, :, :1], rows, rows[:, :, -1:], rows[:, :, -1:]],
        axis=2).reshape(3, TABLE_WORDS)
    chunks = lax.bitcast_convert_type(chunks, jnp.int32)

    out_parts = pl.kernel(
        _sc_body,
        out_type=jax.ShapeDtypeStruct((N_TILES, OUT_FLAT), jnp.int32),
        mesh=plsc.VectorSubcoreMesh(core_axis_name="c", subcore_axis_name="s"),
        compiler_params=pltpu.CompilerParams(
            needs_layout_passes=False, use_tc_tiling_on_sc=False),
        scratch_types=[
            pltpu.VMEM((TABLE_WORDS,), jnp.int32),
            pltpu.VMEM((N_GROUPS, 6, 16), jnp.float32),
            pltpu.VMEM((N_GROUPS, 16), jnp.float32),
            pltpu.VMEM((2 * NUM_SAMPLES + 6, 16), jnp.float32),
            pltpu.VMEM((6, 16), jnp.int32),
            pltpu.VMEM((2 * PAIRS_PER_TILE,), jnp.int32),
        ],
    )(chunks, fields_hbm, tu, lohi, oidx)

    line_map = jnp.sum(out_parts, axis=0)[:N_JUNC * N_JUNC].reshape(N_JUNC, N_JUNC)
    return line_map, junctions, heatmap
